# baseline (speedup 1.0000x reference)
import functools

import jax
import jax.numpy as jnp
from jax.experimental import pallas as pl
from jax.experimental.pallas import tpu as pltpu

N = 10000
E = 160000
D_BLOCK = 256
D_HID = 512
D_Z = 256
HEADS = 4
C = D_HID // HEADS


def _ln(x, g, b):
    mu = x.mean(-1, keepdims=True)
    v = ((x - mu) ** 2).mean(-1, keepdims=True)
    return (x - mu) / jnp.sqrt(v + 1e-5) * g + b


def _out_proj_kernel(x_ref, g_ref, b_ref, w1_ref, b1_ref, w2_ref, b2_ref, o_ref):
    x = x_ref[...]
    mu = jnp.mean(x, axis=-1, keepdims=True)
    var = jnp.mean((x - mu) ** 2, axis=-1, keepdims=True)
    xn = (x - mu) * jax.lax.rsqrt(var + 1e-5) * g_ref[...] + b_ref[...]
    h = jnp.maximum(jnp.dot(xn, w1_ref[...], preferred_element_type=jnp.float32)
                    + b1_ref[...], 0.0)
    o_ref[...] = jnp.dot(h, w2_ref[...], preferred_element_type=jnp.float32) + b2_ref[...]


def _out_proj(x, g, b, w1, b1, w2, b2):
    blk = 1000
    grid = (N // blk,)
    return pl.pallas_call(
        _out_proj_kernel,
        grid=grid,
        in_specs=[
            pl.BlockSpec((blk, D_HID), lambda i: (i, 0)),
            pl.BlockSpec((D_HID,), lambda i: (0,)),
            pl.BlockSpec((D_HID,), lambda i: (0,)),
            pl.BlockSpec((D_HID, D_Z), lambda i: (0, 0)),
            pl.BlockSpec((D_Z,), lambda i: (0,)),
            pl.BlockSpec((D_Z, D_Z), lambda i: (0, 0)),
            pl.BlockSpec((D_Z,), lambda i: (0,)),
        ],
        out_specs=pl.BlockSpec((blk, D_Z), lambda i: (i, 0)),
        out_shape=jax.ShapeDtypeStruct((N, D_Z), jnp.float32),
    )(x, g, b, w1, b1, w2, b2)


def _gatv2(x, src, dst, ep, Wl, bl, Wr, br, att, bo):
    n = x.shape[0]
    ef = src.shape[0]
    xl = x @ Wl + bl
    xr = x @ Wr + br
    xj = xl[src].reshape(ef, HEADS, C)
    xi = xr[dst].reshape(ef, HEADS, C)
    m = jax.nn.leaky_relu(xj + xi + ep.reshape(ef, HEADS, C), 0.2)
    alpha = (m * att).sum(-1)
    amax = jax.ops.segment_max(alpha, dst, num_segments=n)
    ex = jnp.exp(alpha - amax[dst])
    den = jax.ops.segment_sum(ex, dst, num_segments=n)
    a = ex / (den[dst] + 1e-16)
    out = jax.ops.segment_sum(xj * a[:, :, None], dst, num_segments=n)
    return out.reshape(n, HEADS * C) + bo


def kernel(block_features, block_edge_index, block_edge_attr, ln_in_g, ln_in_b,
           W_in, b_in, W_e, b_e, Wl1, bl1, Wr1, br1, We1, att1, bo1, ln1_g, ln1_b,
           Wl2, bl2, Wr2, br2, We2, att2, bo2, ln2_g, ln2_b, Wo1, bo1w, Wo2, bo2w):
    x = _ln(block_features, ln_in_g, ln_in_b)
    x = jax.nn.relu(x @ W_in + b_in)
    e = jax.nn.relu(block_edge_attr @ W_e + b_e)
    src = jnp.concatenate([block_edge_index[0], block_edge_index[1]], axis=0)
    dst = jnp.concatenate([block_edge_index[1], block_edge_index[0]], axis=0)
    e_full = jnp.concatenate([e, e], axis=0)
    ep1 = e_full @ We1
    x = x + _gatv2(x, src, dst, ep1, Wl1, bl1, Wr1, br1, att1, bo1)
    x = _ln(x, ln1_g, ln1_b)
    ep2 = e_full @ We2
    x = x + _gatv2(x, src, dst, ep2, Wl2, bl2, Wr2, br2, att2, bo2)
    return _out_proj(x, ln2_g, ln2_b, Wo1, bo1w, Wo2, bo2w)


# speedup vs baseline: 6.0896x; 6.0896x over previous
import functools

import jax
import jax.numpy as jnp
from jax import lax
from jax.experimental import pallas as pl
from jax.experimental.pallas import tpu as pltpu
from jax.experimental.pallas import tpu_sc as plsc

N = 10000
E = 160000
E2 = 2 * E
D_BLOCK = 256
D_HID = 512
D_Z = 256
HEADS = 4
C = D_HID // HEADS
NCH = D_HID // 16

NC_SC, NS_SC = 2, 16
NW = NC_SC * NS_SC
EPW = E2 // NW
NPW = 320
NPAD = NW * NPW
NHALF = NPW // 2
WCAP = 8192
EW3 = 2048
G = 16
E2P = E2 + WCAP + 16

NEG_INF = float("-inf")

_mesh = plsc.VectorSubcoreMesh(core_axis_name="c", subcore_axis_name="s")


def _wid():
    return lax.axis_index("s") * NC_SC + lax.axis_index("c")


def _store_scalar1(ref1d, j, val):
    plsc.store_scatter(ref1d, [jnp.full((16,), j, jnp.int32)],
                       jnp.full((16,), val),
                       mask=lax.iota(jnp.int32, 16) == 0)


def _hsum16(v):
    s = [v[i] for i in range(16)]
    while len(s) > 1:
        s = [s[2 * i] + s[2 * i + 1] for i in range(len(s) // 2)]
    return s[0]


def _hmax16(v):
    s = [v[i] for i in range(16)]
    while len(s) > 1:
        s = [jnp.maximum(s[2 * i], s[2 * i + 1]) for i in range(len(s) // 2)]
    return s[0]


def _p1_body(xl_hbm, xr_hbm, ep_hbm, src_hbm, dst_hbm, eid_hbm, att_hbm,
             alpha_hbm, srcv, dstv, eidv, attv, xlb, xrb, epb, aloc,
             s1, s2, s3):
    w = _wid()
    e0 = w * EPW
    pltpu.sync_copy(src_hbm.at[pl.ds(e0, EPW)], srcv)
    pltpu.sync_copy(dst_hbm.at[pl.ds(e0, EPW)], dstv)
    pltpu.sync_copy(eid_hbm.at[pl.ds(e0, EPW)], eidv)
    pltpu.sync_copy(att_hbm, attv)
    ri = lax.iota(jnp.int32, 16)

    def chunk(ch, _):
        off = ch * G
        c1 = pltpu.async_copy(xl_hbm.at[srcv.at[pl.ds(off, G)]], xlb, s1)
        c2 = pltpu.async_copy(xr_hbm.at[dstv.at[pl.ds(off, G)]], xrb, s2)
        c3 = pltpu.async_copy(ep_hbm.at[eidv.at[pl.ds(off, G)]], epb, s3)
        c1.wait()
        c2.wait()
        c3.wait()

        def edge(e, res):
            acc = [jnp.zeros((16,), jnp.float32) for _ in range(HEADS)]
            for c in range(NCH):
                sl = pl.ds(c * 16, 16)
                sv = xlb[e, sl] + xrb[e, sl] + epb[e, sl]
                m = jnp.maximum(sv, jnp.float32(0.2) * sv)
                h = c // 8
                av = attv[pl.ds(c * 16, 16)]
                acc[h] = acc[h] + m * av
            out = []
            for h in range(HEADS):
                out.append(jnp.where(ri == e, jnp.full((16,), _hsum16(acc[h])),
                                     res[h]))
            return tuple(out)

        z16 = jnp.zeros((16,), jnp.float32)
        res = lax.fori_loop(0, G, edge, (z16,) * HEADS)
        for h in range(HEADS):
            aloc[pl.ds(h * EPW + off, 16)] = res[h]
        return 0

    lax.fori_loop(0, EPW // G, chunk, 0)
    for h in range(HEADS):
        pltpu.sync_copy(aloc.at[pl.ds(h * EPW, EPW)],
                        alpha_hbm.at[pl.ds(h * E2P + e0, EPW)])


def _p1(xl, xr, ep, srcp, dstp, eidp, att):
    kfn = pl.kernel(
        _p1_body,
        out_type=jax.ShapeDtypeStruct((HEADS * E2P,), jnp.float32),
        mesh=_mesh,
        scratch_types=[
            pltpu.VMEM((EPW,), jnp.int32),
            pltpu.VMEM((EPW,), jnp.int32),
            pltpu.VMEM((EPW,), jnp.int32),
            pltpu.VMEM((D_HID,), jnp.float32),
            pltpu.VMEM((G, D_HID), jnp.float32),
            pltpu.VMEM((G, D_HID), jnp.float32),
            pltpu.VMEM((G, D_HID), jnp.float32),
            pltpu.VMEM((HEADS * EPW,), jnp.float32),
            pltpu.SemaphoreType.DMA,
            pltpu.SemaphoreType.DMA,
            pltpu.SemaphoreType.DMA,
        ],
    )
    return kfn(xl, xr, ep, srcp, dstp, eidp, att)


def _p2_body(alpha_hbm, ro_hbm, amax_hbm, den_hbm, win, am2, dn2, ro_v):
    w = _wid()
    n0 = w * NPW
    pltpu.sync_copy(ro_hbm.at[pl.ds(n0, NPW + 16)], ro_v)
    ri = lax.iota(jnp.int32, 16)

    def head(h, _):
        def group(g, cur_wb):
            rv = ro_v[pl.ds(g * 16, 16)]
            rvn = ro_v[pl.ds(g * 16 + 16, 16)]
            wb_c = cur_wb
            for k in range(16):
                s = rv[k]
                e = rvn[0] if k == 15 else rv[k + 1]
                c0 = s // 16
                c1 = (e + 15) // 16

                def chk(c, carry):
                    m, d, wb = carry
                    nwb = (c * 16) // WCAP * WCAP

                    @pl.when(nwb != wb)
                    def _():
                        pltpu.sync_copy(
                            alpha_hbm.at[pl.ds(h * E2P + nwb, WCAP)], win)

                    loc = c * 16 - nwb
                    a = win[pl.ds(loc, 16)]
                    ids = c * 16 + ri
                    vm = (ids >= s) & (ids < e)
                    am = jnp.where(vm, a, NEG_INF)
                    mn = jnp.maximum(m, am)
                    scale = jnp.where(mn == NEG_INF, jnp.float32(1.0),
                                      jnp.exp(m - mn))
                    t = jnp.where(am == NEG_INF, jnp.float32(0.0),
                                  jnp.exp(am - mn))
                    return (mn, d * scale + t, nwb)

                m0 = jnp.full((16,), NEG_INF, jnp.float32)
                d0 = jnp.zeros((16,), jnp.float32)
                m, d, wb_c = lax.fori_loop(c0, c1, chk, (m0, d0, wb_c))
                mm = _hmax16(m)
                dv2 = jnp.where(m == NEG_INF, jnp.float32(0.0),
                                d * jnp.exp(m - jnp.full((16,), mm)))
                dd = _hsum16(dv2)
                n_loc = g * 16 + k
                am2[pl.ds(n_loc * 16, 16)] = jnp.full((16,), mm)
                dn2[pl.ds(n_loc * 16, 16)] = jnp.full((16,), dd)
            return wb_c

        lax.fori_loop(0, NPW // 16, group, jnp.int32(-(2 ** 30)))
        pltpu.sync_copy(am2,
                        amax_hbm.at[pl.ds((h * NPAD + n0) * 16, NPW * 16)])
        pltpu.sync_copy(dn2,
                        den_hbm.at[pl.ds((h * NPAD + n0) * 16, NPW * 16)])
        return 0

    lax.fori_loop(0, HEADS, head, 0)


def _p2(alpha, ro):
    kfn = pl.kernel(
        _p2_body,
        out_type=(jax.ShapeDtypeStruct((HEADS * NPAD * 16,), jnp.float32),
                  jax.ShapeDtypeStruct((HEADS * NPAD * 16,), jnp.float32)),
        mesh=_mesh,
        scratch_types=[
            pltpu.VMEM((WCAP,), jnp.float32),
            pltpu.VMEM((NPW * 16,), jnp.float32),
            pltpu.VMEM((NPW * 16,), jnp.float32),
            pltpu.VMEM((NPW + 16,), jnp.int32),
        ],
    )
    return kfn(alpha, ro)


NQ = 4
NQROW = NPW // NQ


def _p3_body(xl_hbm, alpha_hbm, amax_hbm, den_hbm, src_hbm, dst_hbm, ro_hbm,
             out_hbm, srcw, dstw, aw, rep_m, rep_d, acc, rows, wbuf, ro_v,
             sg):
    w = _wid()
    n0 = w * NPW
    pltpu.sync_copy(ro_hbm.at[pl.ds(n0, NPW + 16)], ro_v)
    for h in range(HEADS):
        pltpu.sync_copy(amax_hbm.at[pl.ds((h * NPAD + n0) * 16, NPW * 16)],
                        rep_m.at[pl.ds(h * NPW * 16, NPW * 16)])
        pltpu.sync_copy(den_hbm.at[pl.ds((h * NPAD + n0) * 16, NPW * 16)],
                        rep_d.at[pl.ds(h * NPW * 16, NPW * 16)])
    ri = lax.iota(jnp.int32, 16)

    def quarter(q, _):
        nb = n0 + q * NQROW

        def zrow(i, _):
            acc[pl.ds(i * 16, 16)] = jnp.zeros((16,), jnp.float32)
            return 0

        lax.fori_loop(0, NQROW * D_HID // 16, zrow, 0)
        s_h = ro_v[pl.ds(q * NQROW, 16)][0]
        e_h = ro_v[pl.ds(q * NQROW + NQROW, 16)][0]
        base = s_h // 16 * 16
        nch = (e_h - base + 15) // 16

        def chk(j, _):
            goff = base + j * 16

            @pl.when(j % (EW3 // 16) == 0)
            def _():
                pltpu.sync_copy(src_hbm.at[pl.ds(goff, EW3)], srcw)
                pltpu.sync_copy(dst_hbm.at[pl.ds(goff, EW3)], dstw)
                for hh in range(HEADS):
                    pltpu.sync_copy(alpha_hbm.at[pl.ds(hh * E2P + goff, EW3)],
                                    aw.at[pl.ds(hh * EW3, EW3)])

            loc = j % (EW3 // 16) * 16
            pltpu.async_copy(xl_hbm.at[srcw.at[pl.ds(loc, G)]], rows,
                             sg).wait()
            dstv = dstw[pl.ds(loc, 16)]
            avs = [aw[pl.ds(h * EW3 + loc, 16)] for h in range(HEADS)]
            z16 = jnp.zeros((16,), jnp.float32)
            tv = [z16] * HEADS
            dv = [jnp.full((16,), jnp.float32(1.0))] * HEADS
            for e in range(G):
                dl = jnp.clip(dstv[e] - n0, 0, NPW - 1)
                lane = ri == e
                for h in range(HEADS):
                    mrow = rep_m[pl.ds((h * NPW + dl) * 16, 16)]
                    drow = rep_d[pl.ds((h * NPW + dl) * 16, 16)]
                    tv[h] = jnp.where(lane,
                                      jnp.full((16,), avs[h][e] - mrow[0]),
                                      tv[h])
                    dv[h] = jnp.where(lane, jnp.full((16,), drow[0]), dv[h])
            for h in range(HEADS):
                wbuf[pl.ds(h * 16, 16)] = (
                    jnp.exp(tv[h]) / (dv[h] + jnp.float32(1e-16)))
            for e in range(G):
                gid = goff + e

                @pl.when((gid >= s_h) & (gid < e_h))
                def _(e=e):
                    da = dstv[e] - nb

                    def hloop(h, _):
                        wvec = wbuf[pl.ds(h * 16, 16)]
                        wf = jnp.full((16,), wvec[e])
                        for cc in range(C // 16):
                            off = da * D_HID + h * C + cc * 16
                            plsc.addupdate(
                                acc.at[pl.ds(off, 16)],
                                wf * rows[e, pl.ds(h * C + cc * 16, 16)])
                        return 0

                    lax.fori_loop(0, HEADS, hloop, 0)
            return 0

        lax.fori_loop(0, nch, chk, 0)
        pltpu.sync_copy(
            acc, out_hbm.at[pl.ds(nb * D_HID, NQROW * D_HID)])
        return 0

    lax.fori_loop(0, NQ, quarter, 0)


def _p3(xl, alpha, amax, den, srcp, dstp, ro):
    kfn = pl.kernel(
        _p3_body,
        out_type=jax.ShapeDtypeStruct((NPAD * D_HID,), jnp.float32),
        mesh=_mesh,
        scratch_types=[
            pltpu.VMEM((EW3,), jnp.int32),
            pltpu.VMEM((EW3,), jnp.int32),
            pltpu.VMEM((HEADS * EW3,), jnp.float32),
            pltpu.VMEM((HEADS * NPW * 16,), jnp.float32),
            pltpu.VMEM((HEADS * NPW * 16,), jnp.float32),
            pltpu.VMEM((NQROW * D_HID,), jnp.float32),
            pltpu.VMEM((G, D_HID), jnp.float32),
            pltpu.VMEM((HEADS * 16,), jnp.float32),
            pltpu.VMEM((NPW + 16,), jnp.int32),
            pltpu.SemaphoreType.DMA,
        ],
    )
    return kfn(xl, alpha, amax, den, srcp, dstp, ro)


def _ln_block(x, g, b):
    mu = jnp.mean(x, axis=-1, keepdims=True)
    var = jnp.mean((x - mu) ** 2, axis=-1, keepdims=True)
    return (x - mu) * lax.rsqrt(var + 1e-5) * g + b


def _tc_pre_body(bf, g, b, Win, bin_, Wl, bl, Wr, br, x0o, xlo, xro):
    xn = _ln_block(bf[...], g[...], b[...])
    x0 = jnp.maximum(
        jnp.dot(xn, Win[...], preferred_element_type=jnp.float32) + bin_[...],
        0.0)
    x0o[...] = x0
    xlo[...] = jnp.dot(x0, Wl[...], preferred_element_type=jnp.float32) + bl[...]
    xro[...] = jnp.dot(x0, Wr[...], preferred_element_type=jnp.float32) + br[...]


def _tc_pre(bf, g, b, Win, bin_, Wl, bl, Wr, br):
    blk = 1000
    vec = lambda dim: pl.BlockSpec((dim,), lambda i: (0,))
    mat = lambda r, c: pl.BlockSpec((r, c), lambda i: (0, 0))
    out = jax.ShapeDtypeStruct((N, D_HID), jnp.float32)
    return pl.pallas_call(
        _tc_pre_body,
        grid=(N // blk,),
        in_specs=[
            pl.BlockSpec((blk, D_BLOCK), lambda i: (i, 0)),
            vec(D_BLOCK), vec(D_BLOCK),
            mat(D_BLOCK, D_HID), vec(D_HID),
            mat(D_HID, D_HID), vec(D_HID),
            mat(D_HID, D_HID), vec(D_HID),
        ],
        out_specs=[pl.BlockSpec((blk, D_HID), lambda i: (i, 0))] * 3,
        out_shape=[out, out, out],
    )(bf, g, b, Win, bin_, Wl, bl, Wr, br)


def _tc_ew_body(ea, We, be, eo):
    eo[...] = jnp.maximum(
        jnp.dot(ea[...], We[...], preferred_element_type=jnp.float32) + be[...],
        0.0)


def _tc_ew(ea, We, be):
    blk = 4000
    return pl.pallas_call(
        _tc_ew_body,
        grid=(E // blk,),
        in_specs=[
            pl.BlockSpec((blk, 16), lambda i: (i, 0)),
            pl.BlockSpec((16, D_HID), lambda i: (0, 0)),
            pl.BlockSpec((D_HID,), lambda i: (0,)),
        ],
        out_specs=pl.BlockSpec((blk, D_HID), lambda i: (i, 0)),
        out_shape=jax.ShapeDtypeStruct((E, D_HID), jnp.float32),
    )(ea, We, be)


def _tc_ep_body(ew, W1, W2, o1, o2):
    e = ew[...]
    o1[...] = jnp.dot(e, W1[...], preferred_element_type=jnp.float32)
    o2[...] = jnp.dot(e, W2[...], preferred_element_type=jnp.float32)


def _tc_ep(ew, W1, W2):
    blk = 2000
    out = jax.ShapeDtypeStruct((E, D_HID), jnp.float32)
    return pl.pallas_call(
        _tc_ep_body,
        grid=(E // blk,),
        in_specs=[
            pl.BlockSpec((blk, D_HID), lambda i: (i, 0)),
            pl.BlockSpec((D_HID, D_HID), lambda i: (0, 0)),
            pl.BlockSpec((D_HID, D_HID), lambda i: (0, 0)),
        ],
        out_specs=[pl.BlockSpec((blk, D_HID), lambda i: (i, 0))] * 2,
        out_shape=[out, out],
    )(ew, W1, W2)


def _tc_mid_body(x, msg, bo, g, b, Wl, bl, Wr, br, x1o, xlo, xro):
    xn = _ln_block(x[...] + msg[...] + bo[...], g[...], b[...])
    x1o[...] = xn
    xlo[...] = jnp.dot(xn, Wl[...], preferred_element_type=jnp.float32) + bl[...]
    xro[...] = jnp.dot(xn, Wr[...], preferred_element_type=jnp.float32) + br[...]


def _tc_mid(x, msg, bo, g, b, Wl, bl, Wr, br):
    blk = 1000
    vec = lambda dim: pl.BlockSpec((dim,), lambda i: (0,))
    mat = lambda r, c: pl.BlockSpec((r, c), lambda i: (0, 0))
    out = jax.ShapeDtypeStruct((N, D_HID), jnp.float32)
    return pl.pallas_call(
        _tc_mid_body,
        grid=(N // blk,),
        in_specs=[
            pl.BlockSpec((blk, D_HID), lambda i: (i, 0)),
            pl.BlockSpec((blk, D_HID), lambda i: (i, 0)),
            vec(D_HID), vec(D_HID), vec(D_HID),
            mat(D_HID, D_HID), vec(D_HID),
            mat(D_HID, D_HID), vec(D_HID),
        ],
        out_specs=[pl.BlockSpec((blk, D_HID), lambda i: (i, 0))] * 3,
        out_shape=[out, out, out],
    )(x, msg, bo, g, b, Wl, bl, Wr, br)


def _tc_out_body(x, msg, bo, g, b, W1, b1, W2, b2, zo):
    xn = _ln_block(x[...] + msg[...] + bo[...], g[...], b[...])
    hh = jnp.maximum(
        jnp.dot(xn, W1[...], preferred_element_type=jnp.float32) + b1[...],
        0.0)
    zo[...] = jnp.dot(hh, W2[...], preferred_element_type=jnp.float32) + b2[...]


def _tc_out(x, msg, bo, g, b, W1, b1, W2, b2):
    blk = 1000
    vec = lambda dim: pl.BlockSpec((dim,), lambda i: (0,))
    mat = lambda r, c: pl.BlockSpec((r, c), lambda i: (0, 0))
    return pl.pallas_call(
        _tc_out_body,
        grid=(N // blk,),
        in_specs=[
            pl.BlockSpec((blk, D_HID), lambda i: (i, 0)),
            pl.BlockSpec((blk, D_HID), lambda i: (i, 0)),
            vec(D_HID), vec(D_HID), vec(D_HID),
            mat(D_HID, D_Z), vec(D_Z),
            mat(D_Z, D_Z), vec(D_Z),
        ],
        out_specs=pl.BlockSpec((blk, D_Z), lambda i: (i, 0)),
        out_shape=jax.ShapeDtypeStruct((N, D_Z), jnp.float32),
    )(x, msg, bo, g, b, W1, b1, W2, b2)


def kernel(block_features, block_edge_index, block_edge_attr, ln_in_g,
           ln_in_b, W_in, b_in, W_e, b_e, Wl1, bl1, Wr1, br1, We1, att1, bo1,
           ln1_g, ln1_b, Wl2, bl2, Wr2, br2, We2, att2, bo2, ln2_g, ln2_b,
           Wo1, bo1w, Wo2, bo2w):
    ei = block_edge_index.astype(jnp.int32)
    s, d = ei[0], ei[1]
    src2 = jnp.concatenate([s, d])
    dst2 = jnp.concatenate([d, s])
    eid = jnp.arange(E, dtype=jnp.int32)
    eid2 = jnp.concatenate([eid, eid])
    perm = jnp.argsort(dst2)
    ssrc = src2[perm]
    sdst = dst2[perm]
    seid = eid2[perm]
    pad = E2P - E2
    ssrc_p = jnp.pad(ssrc, (0, pad))
    sdst_p = jnp.pad(sdst, (0, pad))
    seid_p = jnp.pad(seid, (0, pad))
    ro = jnp.searchsorted(sdst, jnp.arange(NPAD + 1, dtype=jnp.int32),
                          side="left").astype(jnp.int32)
    ro_p = jnp.pad(ro, (0, 15))

    x0, xl1, xr1 = _tc_pre(block_features, ln_in_g, ln_in_b, W_in, b_in,
                           Wl1, bl1, Wr1, br1)
    ew = _tc_ew(block_edge_attr, W_e, b_e)
    ep1, ep2 = _tc_ep(ew, We1, We2)

    att1f = jnp.reshape(att1, (D_HID,))
    att2f = jnp.reshape(att2, (D_HID,))
    al1 = _p1(xl1, xr1, ep1, ssrc_p, sdst_p, seid_p, att1f)
    am1, dn1 = _p2(al1, ro_p)
    msg1 = jnp.reshape(_p3(xl1, al1, am1, dn1, ssrc_p, sdst_p,
                           ro_p), (NPAD, D_HID))

    x1, xl2, xr2 = _tc_mid(x0, msg1, bo1, ln1_g, ln1_b, Wl2, bl2, Wr2, br2)

    al2 = _p1(xl2, xr2, ep2, ssrc_p, sdst_p, seid_p, att2f)
    am2, dn2 = _p2(al2, ro_p)
    msg2 = jnp.reshape(_p3(xl2, al2, am2, dn2, ssrc_p, sdst_p,
                           ro_p), (NPAD, D_HID))

    return _tc_out(x1, msg2, bo2, ln2_g, ln2_b, Wo1, bo1w, Wo2, bo2w)


# speedup vs baseline: 6.9170x; 1.1359x over previous
import functools

import jax
import jax.numpy as jnp
from jax import lax
from jax.experimental import pallas as pl
from jax.experimental.pallas import tpu as pltpu
from jax.experimental.pallas import tpu_sc as plsc

N = 10000
E = 160000
E2 = 2 * E
D_BLOCK = 256
D_HID = 512
D_Z = 256
HEADS = 4
C = D_HID // HEADS
NCH = D_HID // 16

NC_SC, NS_SC = 2, 16
NW = NC_SC * NS_SC
EPW = E2 // NW
NPW = 320
NPAD = NW * NPW
NHALF = NPW // 2
WCAP = 8192
EW3 = 2048
G = 16
E2P = E2 + WCAP + 16

NEG_INF = float("-inf")

_mesh = plsc.VectorSubcoreMesh(core_axis_name="c", subcore_axis_name="s")


def _wid():
    return lax.axis_index("s") * NC_SC + lax.axis_index("c")


def _store_scalar1(ref1d, j, val):
    plsc.store_scatter(ref1d, [jnp.full((16,), j, jnp.int32)],
                       jnp.full((16,), val),
                       mask=lax.iota(jnp.int32, 16) == 0)


def _hsum16(v):
    s = [v[i] for i in range(16)]
    while len(s) > 1:
        s = [s[2 * i] + s[2 * i + 1] for i in range(len(s) // 2)]
    return s[0]


def _hmax16(v):
    s = [v[i] for i in range(16)]
    while len(s) > 1:
        s = [jnp.maximum(s[2 * i], s[2 * i + 1]) for i in range(len(s) // 2)]
    return s[0]


def _p1_body(xl_hbm, xr_hbm, ep_hbm, src_hbm, dst_hbm, eid_hbm, att_hbm,
             alpha_hbm, srcv, dstv, eidv, attv, xlb0, xrb0, epb0, xlb1, xrb1,
             epb1, aloc, s10, s20, s30, s11, s21, s31):
    w = _wid()
    e0 = w * EPW
    pltpu.sync_copy(src_hbm.at[pl.ds(e0, EPW + 2 * G)], srcv)
    pltpu.sync_copy(dst_hbm.at[pl.ds(e0, EPW + 2 * G)], dstv)
    pltpu.sync_copy(eid_hbm.at[pl.ds(e0, EPW + 2 * G)], eidv)
    pltpu.sync_copy(att_hbm, attv)
    ri = lax.iota(jnp.int32, 16)
    bufs = ((xlb0, xrb0, epb0, s10, s20, s30),
            (xlb1, xrb1, epb1, s11, s21, s31))

    def issue(ch, bs):
        xlb, xrb, epb, s1, s2, s3 = bs
        off = ch * G
        pltpu.async_copy(xl_hbm.at[srcv.at[pl.ds(off, G)]], xlb, s1)
        pltpu.async_copy(xr_hbm.at[dstv.at[pl.ds(off, G)]], xrb, s2)
        pltpu.async_copy(ep_hbm.at[eidv.at[pl.ds(off, G)]], epb, s3)

    def process(ch, bs):
        xlb, xrb, epb, s1, s2, s3 = bs
        pltpu.make_async_copy(xl_hbm.at[srcv.at[pl.ds(0, G)]], xlb, s1).wait()
        pltpu.make_async_copy(xr_hbm.at[dstv.at[pl.ds(0, G)]], xrb, s2).wait()
        pltpu.make_async_copy(ep_hbm.at[eidv.at[pl.ds(0, G)]], epb, s3).wait()
        off = ch * G

        def edge(e, res):
            acc = [jnp.zeros((16,), jnp.float32) for _ in range(HEADS)]
            for c in range(NCH):
                sl = pl.ds(c * 16, 16)
                sv = xlb[e, sl] + xrb[e, sl] + epb[e, sl]
                m = jnp.maximum(sv, jnp.float32(0.2) * sv)
                h = c // 8
                av = attv[pl.ds(c * 16, 16)]
                acc[h] = acc[h] + m * av
            out = []
            for h in range(HEADS):
                out.append(jnp.where(ri == e, jnp.full((16,), _hsum16(acc[h])),
                                     res[h]))
            return tuple(out)

        z16 = jnp.zeros((16,), jnp.float32)
        res = lax.fori_loop(0, G, edge, (z16,) * HEADS)
        for h in range(HEADS):
            aloc[pl.ds(h * EPW + off, 16)] = res[h]

    issue(0, bufs[0])

    def step(t, _):
        issue(2 * t + 1, bufs[1])
        process(2 * t, bufs[0])
        issue(2 * t + 2, bufs[0])
        process(2 * t + 1, bufs[1])
        return 0

    lax.fori_loop(0, EPW // (2 * G), step, 0)
    process(EPW // G - 1, bufs[0])
    for h in range(HEADS):
        pltpu.sync_copy(aloc.at[pl.ds(h * EPW, EPW)],
                        alpha_hbm.at[pl.ds(h * E2P + e0, EPW)])


def _p1(xl, xr, ep, srcp, dstp, eidp, att):
    kfn = pl.kernel(
        _p1_body,
        out_type=jax.ShapeDtypeStruct((HEADS * E2P,), jnp.float32),
        mesh=_mesh,
        scratch_types=[
            pltpu.VMEM((EPW + 2 * G,), jnp.int32),
            pltpu.VMEM((EPW + 2 * G,), jnp.int32),
            pltpu.VMEM((EPW + 2 * G,), jnp.int32),
            pltpu.VMEM((D_HID,), jnp.float32),
            pltpu.VMEM((G, D_HID), jnp.float32),
            pltpu.VMEM((G, D_HID), jnp.float32),
            pltpu.VMEM((G, D_HID), jnp.float32),
            pltpu.VMEM((G, D_HID), jnp.float32),
            pltpu.VMEM((G, D_HID), jnp.float32),
            pltpu.VMEM((G, D_HID), jnp.float32),
            pltpu.VMEM((HEADS * EPW,), jnp.float32),
            pltpu.SemaphoreType.DMA,
            pltpu.SemaphoreType.DMA,
            pltpu.SemaphoreType.DMA,
            pltpu.SemaphoreType.DMA,
            pltpu.SemaphoreType.DMA,
            pltpu.SemaphoreType.DMA,
        ],
    )
    return kfn(xl, xr, ep, srcp, dstp, eidp, att)


def _p2_body(alpha_hbm, ro_hbm, amax_hbm, den_hbm, win, am2, dn2, ro_v):
    w = _wid()
    n0 = w * NPW
    pltpu.sync_copy(ro_hbm.at[pl.ds(n0, NPW + 16)], ro_v)
    ri = lax.iota(jnp.int32, 16)

    def head(h, _):
        def group(g, cur_wb):
            rv = ro_v[pl.ds(g * 16, 16)]
            rvn = ro_v[pl.ds(g * 16 + 16, 16)]
            wb_c = cur_wb
            for k in range(16):
                s = rv[k]
                e = rvn[0] if k == 15 else rv[k + 1]
                c0 = s // 16
                c1 = (e + 15) // 16

                def chk(c, carry):
                    m, d, wb = carry
                    nwb = (c * 16) // WCAP * WCAP

                    @pl.when(nwb != wb)
                    def _():
                        pltpu.sync_copy(
                            alpha_hbm.at[pl.ds(h * E2P + nwb, WCAP)], win)

                    loc = c * 16 - nwb
                    a = win[pl.ds(loc, 16)]
                    ids = c * 16 + ri
                    vm = (ids >= s) & (ids < e)
                    am = jnp.where(vm, a, NEG_INF)
                    mn = jnp.maximum(m, am)
                    scale = jnp.where(mn == NEG_INF, jnp.float32(1.0),
                                      jnp.exp(m - mn))
                    t = jnp.where(am == NEG_INF, jnp.float32(0.0),
                                  jnp.exp(am - mn))
                    return (mn, d * scale + t, nwb)

                m0 = jnp.full((16,), NEG_INF, jnp.float32)
                d0 = jnp.zeros((16,), jnp.float32)
                m, d, wb_c = lax.fori_loop(c0, c1, chk, (m0, d0, wb_c))
                mm = _hmax16(m)
                dv2 = jnp.where(m == NEG_INF, jnp.float32(0.0),
                                d * jnp.exp(m - jnp.full((16,), mm)))
                dd = _hsum16(dv2)
                n_loc = g * 16 + k
                am2[pl.ds(n_loc * 16, 16)] = jnp.full((16,), mm)
                dn2[pl.ds(n_loc * 16, 16)] = jnp.full((16,), dd)
            return wb_c

        lax.fori_loop(0, NPW // 16, group, jnp.int32(-(2 ** 30)))
        pltpu.sync_copy(am2,
                        amax_hbm.at[pl.ds((h * NPAD + n0) * 16, NPW * 16)])
        pltpu.sync_copy(dn2,
                        den_hbm.at[pl.ds((h * NPAD + n0) * 16, NPW * 16)])
        return 0

    lax.fori_loop(0, HEADS, head, 0)


def _p2(alpha, ro):
    kfn = pl.kernel(
        _p2_body,
        out_type=(jax.ShapeDtypeStruct((HEADS * NPAD * 16,), jnp.float32),
                  jax.ShapeDtypeStruct((HEADS * NPAD * 16,), jnp.float32)),
        mesh=_mesh,
        scratch_types=[
            pltpu.VMEM((WCAP,), jnp.float32),
            pltpu.VMEM((NPW * 16,), jnp.float32),
            pltpu.VMEM((NPW * 16,), jnp.float32),
            pltpu.VMEM((NPW + 16,), jnp.int32),
        ],
    )
    return kfn(alpha, ro)


NQ = 4
NQROW = NPW // NQ


def _p3_body(xl_hbm, alpha_hbm, amax_hbm, den_hbm, src_hbm, dst_hbm, ro_hbm,
             out_hbm, srcw, dstw, aw, rep_m, rep_d, acc, rows, wbuf, ro_v,
             sg):
    w = _wid()
    n0 = w * NPW
    pltpu.sync_copy(ro_hbm.at[pl.ds(n0, NPW + 16)], ro_v)
    for h in range(HEADS):
        pltpu.sync_copy(amax_hbm.at[pl.ds((h * NPAD + n0) * 16, NPW * 16)],
                        rep_m.at[pl.ds(h * NPW * 16, NPW * 16)])
        pltpu.sync_copy(den_hbm.at[pl.ds((h * NPAD + n0) * 16, NPW * 16)],
                        rep_d.at[pl.ds(h * NPW * 16, NPW * 16)])
    ri = lax.iota(jnp.int32, 16)

    def quarter(q, _):
        nb = n0 + q * NQROW

        def zrow(i, _):
            acc[pl.ds(i * 16, 16)] = jnp.zeros((16,), jnp.float32)
            return 0

        lax.fori_loop(0, NQROW * D_HID // 16, zrow, 0)
        s_h = ro_v[pl.ds(q * NQROW, 16)][0]
        e_h = ro_v[pl.ds(q * NQROW + NQROW, 16)][0]
        base = s_h // 16 * 16
        nch = (e_h - base + 15) // 16

        def chk(j, _):
            goff = base + j * 16

            @pl.when(j % (EW3 // 16) == 0)
            def _():
                pltpu.sync_copy(src_hbm.at[pl.ds(goff, EW3)], srcw)
                pltpu.sync_copy(dst_hbm.at[pl.ds(goff, EW3)], dstw)
                for hh in range(HEADS):
                    pltpu.sync_copy(alpha_hbm.at[pl.ds(hh * E2P + goff, EW3)],
                                    aw.at[pl.ds(hh * EW3, EW3)])

            loc = j % (EW3 // 16) * 16
            pltpu.async_copy(xl_hbm.at[srcw.at[pl.ds(loc, G)]], rows,
                             sg).wait()
            dstv = dstw[pl.ds(loc, 16)]
            avs = [aw[pl.ds(h * EW3 + loc, 16)] for h in range(HEADS)]
            z16 = jnp.zeros((16,), jnp.float32)
            tv = [z16] * HEADS
            dv = [jnp.full((16,), jnp.float32(1.0))] * HEADS
            for e in range(G):
                dl = jnp.clip(dstv[e] - n0, 0, NPW - 1)
                lane = ri == e
                for h in range(HEADS):
                    mrow = rep_m[pl.ds((h * NPW + dl) * 16, 16)]
                    drow = rep_d[pl.ds((h * NPW + dl) * 16, 16)]
                    tv[h] = jnp.where(lane,
                                      jnp.full((16,), avs[h][e] - mrow[0]),
                                      tv[h])
                    dv[h] = jnp.where(lane, jnp.full((16,), drow[0]), dv[h])
            for h in range(HEADS):
                wbuf[pl.ds(h * 16, 16)] = (
                    jnp.exp(tv[h]) / (dv[h] + jnp.float32(1e-16)))
            for e in range(G):
                gid = goff + e

                @pl.when((gid >= s_h) & (gid < e_h))
                def _(e=e):
                    da = dstv[e] - nb

                    def hloop(h, _):
                        wvec = wbuf[pl.ds(h * 16, 16)]
                        wf = jnp.full((16,), wvec[e])
                        for cc in range(C // 16):
                            off = da * D_HID + h * C + cc * 16
                            plsc.addupdate(
                                acc.at[pl.ds(off, 16)],
                                wf * rows[e, pl.ds(h * C + cc * 16, 16)])
                        return 0

                    lax.fori_loop(0, HEADS, hloop, 0)
            return 0

        lax.fori_loop(0, nch, chk, 0)
        pltpu.sync_copy(
            acc, out_hbm.at[pl.ds(nb * D_HID, NQROW * D_HID)])
        return 0

    lax.fori_loop(0, NQ, quarter, 0)


def _p3(xl, alpha, amax, den, srcp, dstp, ro):
    kfn = pl.kernel(
        _p3_body,
        out_type=jax.ShapeDtypeStruct((NPAD * D_HID,), jnp.float32),
        mesh=_mesh,
        scratch_types=[
            pltpu.VMEM((EW3,), jnp.int32),
            pltpu.VMEM((EW3,), jnp.int32),
            pltpu.VMEM((HEADS * EW3,), jnp.float32),
            pltpu.VMEM((HEADS * NPW * 16,), jnp.float32),
            pltpu.VMEM((HEADS * NPW * 16,), jnp.float32),
            pltpu.VMEM((NQROW * D_HID,), jnp.float32),
            pltpu.VMEM((G, D_HID), jnp.float32),
            pltpu.VMEM((HEADS * 16,), jnp.float32),
            pltpu.VMEM((NPW + 16,), jnp.int32),
            pltpu.SemaphoreType.DMA,
        ],
    )
    return kfn(xl, alpha, amax, den, srcp, dstp, ro)


def _ln_block(x, g, b):
    mu = jnp.mean(x, axis=-1, keepdims=True)
    var = jnp.mean((x - mu) ** 2, axis=-1, keepdims=True)
    return (x - mu) * lax.rsqrt(var + 1e-5) * g + b


def _tc_pre_body(bf, g, b, Win, bin_, Wl, bl, Wr, br, x0o, xlo, xro):
    xn = _ln_block(bf[...], g[...], b[...])
    x0 = jnp.maximum(
        jnp.dot(xn, Win[...], preferred_element_type=jnp.float32) + bin_[...],
        0.0)
    x0o[...] = x0
    xlo[...] = jnp.dot(x0, Wl[...], preferred_element_type=jnp.float32) + bl[...]
    xro[...] = jnp.dot(x0, Wr[...], preferred_element_type=jnp.float32) + br[...]


def _tc_pre(bf, g, b, Win, bin_, Wl, bl, Wr, br):
    blk = 1000
    vec = lambda dim: pl.BlockSpec((dim,), lambda i: (0,))
    mat = lambda r, c: pl.BlockSpec((r, c), lambda i: (0, 0))
    out = jax.ShapeDtypeStruct((N, D_HID), jnp.float32)
    return pl.pallas_call(
        _tc_pre_body,
        grid=(N // blk,),
        in_specs=[
            pl.BlockSpec((blk, D_BLOCK), lambda i: (i, 0)),
            vec(D_BLOCK), vec(D_BLOCK),
            mat(D_BLOCK, D_HID), vec(D_HID),
            mat(D_HID, D_HID), vec(D_HID),
            mat(D_HID, D_HID), vec(D_HID),
        ],
        out_specs=[pl.BlockSpec((blk, D_HID), lambda i: (i, 0))] * 3,
        out_shape=[out, out, out],
    )(bf, g, b, Win, bin_, Wl, bl, Wr, br)


def _tc_ew_body(ea, We, be, eo):
    eo[...] = jnp.maximum(
        jnp.dot(ea[...], We[...], preferred_element_type=jnp.float32) + be[...],
        0.0)


def _tc_ew(ea, We, be):
    blk = 4000
    return pl.pallas_call(
        _tc_ew_body,
        grid=(E // blk,),
        in_specs=[
            pl.BlockSpec((blk, 16), lambda i: (i, 0)),
            pl.BlockSpec((16, D_HID), lambda i: (0, 0)),
            pl.BlockSpec((D_HID,), lambda i: (0,)),
        ],
        out_specs=pl.BlockSpec((blk, D_HID), lambda i: (i, 0)),
        out_shape=jax.ShapeDtypeStruct((E, D_HID), jnp.float32),
    )(ea, We, be)


def _tc_ep_body(ew, W1, W2, o1, o2):
    e = ew[...]
    o1[...] = jnp.dot(e, W1[...], preferred_element_type=jnp.float32)
    o2[...] = jnp.dot(e, W2[...], preferred_element_type=jnp.float32)


def _tc_ep(ew, W1, W2):
    blk = 2000
    out = jax.ShapeDtypeStruct((E, D_HID), jnp.float32)
    return pl.pallas_call(
        _tc_ep_body,
        grid=(E // blk,),
        in_specs=[
            pl.BlockSpec((blk, D_HID), lambda i: (i, 0)),
            pl.BlockSpec((D_HID, D_HID), lambda i: (0, 0)),
            pl.BlockSpec((D_HID, D_HID), lambda i: (0, 0)),
        ],
        out_specs=[pl.BlockSpec((blk, D_HID), lambda i: (i, 0))] * 2,
        out_shape=[out, out],
    )(ew, W1, W2)


def _tc_mid_body(x, msg, bo, g, b, Wl, bl, Wr, br, x1o, xlo, xro):
    xn = _ln_block(x[...] + msg[...] + bo[...], g[...], b[...])
    x1o[...] = xn
    xlo[...] = jnp.dot(xn, Wl[...], preferred_element_type=jnp.float32) + bl[...]
    xro[...] = jnp.dot(xn, Wr[...], preferred_element_type=jnp.float32) + br[...]


def _tc_mid(x, msg, bo, g, b, Wl, bl, Wr, br):
    blk = 1000
    vec = lambda dim: pl.BlockSpec((dim,), lambda i: (0,))
    mat = lambda r, c: pl.BlockSpec((r, c), lambda i: (0, 0))
    out = jax.ShapeDtypeStruct((N, D_HID), jnp.float32)
    return pl.pallas_call(
        _tc_mid_body,
        grid=(N // blk,),
        in_specs=[
            pl.BlockSpec((blk, D_HID), lambda i: (i, 0)),
            pl.BlockSpec((blk, D_HID), lambda i: (i, 0)),
            vec(D_HID), vec(D_HID), vec(D_HID),
            mat(D_HID, D_HID), vec(D_HID),
            mat(D_HID, D_HID), vec(D_HID),
        ],
        out_specs=[pl.BlockSpec((blk, D_HID), lambda i: (i, 0))] * 3,
        out_shape=[out, out, out],
    )(x, msg, bo, g, b, Wl, bl, Wr, br)


def _tc_out_body(x, msg, bo, g, b, W1, b1, W2, b2, zo):
    xn = _ln_block(x[...] + msg[...] + bo[...], g[...], b[...])
    hh = jnp.maximum(
        jnp.dot(xn, W1[...], preferred_element_type=jnp.float32) + b1[...],
        0.0)
    zo[...] = jnp.dot(hh, W2[...], preferred_element_type=jnp.float32) + b2[...]


def _tc_out(x, msg, bo, g, b, W1, b1, W2, b2):
    blk = 1000
    vec = lambda dim: pl.BlockSpec((dim,), lambda i: (0,))
    mat = lambda r, c: pl.BlockSpec((r, c), lambda i: (0, 0))
    return pl.pallas_call(
        _tc_out_body,
        grid=(N // blk,),
        in_specs=[
            pl.BlockSpec((blk, D_HID), lambda i: (i, 0)),
            pl.BlockSpec((blk, D_HID), lambda i: (i, 0)),
            vec(D_HID), vec(D_HID), vec(D_HID),
            mat(D_HID, D_Z), vec(D_Z),
            mat(D_Z, D_Z), vec(D_Z),
        ],
        out_specs=pl.BlockSpec((blk, D_Z), lambda i: (i, 0)),
        out_shape=jax.ShapeDtypeStruct((N, D_Z), jnp.float32),
    )(x, msg, bo, g, b, W1, b1, W2, b2)


def kernel(block_features, block_edge_index, block_edge_attr, ln_in_g,
           ln_in_b, W_in, b_in, W_e, b_e, Wl1, bl1, Wr1, br1, We1, att1, bo1,
           ln1_g, ln1_b, Wl2, bl2, Wr2, br2, We2, att2, bo2, ln2_g, ln2_b,
           Wo1, bo1w, Wo2, bo2w):
    ei = block_edge_index.astype(jnp.int32)
    s, d = ei[0], ei[1]
    src2 = jnp.concatenate([s, d])
    dst2 = jnp.concatenate([d, s])
    eid = jnp.arange(E, dtype=jnp.int32)
    eid2 = jnp.concatenate([eid, eid])
    perm = jnp.argsort(dst2)
    ssrc = src2[perm]
    sdst = dst2[perm]
    seid = eid2[perm]
    pad = E2P - E2
    ssrc_p = jnp.pad(ssrc, (0, pad))
    sdst_p = jnp.pad(sdst, (0, pad))
    seid_p = jnp.pad(seid, (0, pad))
    ro = jnp.searchsorted(sdst, jnp.arange(NPAD + 1, dtype=jnp.int32),
                          side="left").astype(jnp.int32)
    ro_p = jnp.pad(ro, (0, 15))

    x0, xl1, xr1 = _tc_pre(block_features, ln_in_g, ln_in_b, W_in, b_in,
                           Wl1, bl1, Wr1, br1)
    ew = _tc_ew(block_edge_attr, W_e, b_e)
    ep1, ep2 = _tc_ep(ew, We1, We2)

    att1f = jnp.reshape(att1, (D_HID,))
    att2f = jnp.reshape(att2, (D_HID,))
    al1 = _p1(xl1, xr1, ep1, ssrc_p, sdst_p, seid_p, att1f)
    am1, dn1 = _p2(al1, ro_p)
    msg1 = jnp.reshape(_p3(xl1, al1, am1, dn1, ssrc_p, sdst_p,
                           ro_p), (NPAD, D_HID))

    x1, xl2, xr2 = _tc_mid(x0, msg1, bo1, ln1_g, ln1_b, Wl2, bl2, Wr2, br2)

    al2 = _p1(xl2, xr2, ep2, ssrc_p, sdst_p, seid_p, att2f)
    am2, dn2 = _p2(al2, ro_p)
    msg2 = jnp.reshape(_p3(xl2, al2, am2, dn2, ssrc_p, sdst_p,
                           ro_p), (NPAD, D_HID))

    return _tc_out(x1, msg2, bo2, ln2_g, ln2_b, Wo1, bo1w, Wo2, bo2w)


# speedup vs baseline: 6.9314x; 1.0021x over previous
import jax
import jax.numpy as jnp
from jax import lax
from jax.experimental import pallas as pl
from jax.experimental.pallas import tpu as pltpu
from jax.experimental.pallas import tpu_sc as plsc

N = 10000
E = 160000
E2 = 2 * E
D_BLOCK = 256
D_HID = 512
D_Z = 256
HEADS = 4
C = D_HID // HEADS
NCH = D_HID // 16

NC_SC, NS_SC = 2, 16
NW = NC_SC * NS_SC
EPW = E2 // NW
NPW = 320
NPAD = NW * NPW
WCAP = 8192
EW3 = 2048
G = 16
E2P = E2 + WCAP + 16

NEG_INF = float("-inf")

_mesh = plsc.VectorSubcoreMesh(core_axis_name="c", subcore_axis_name="s")


def _wid():
    return lax.axis_index("s") * NC_SC + lax.axis_index("c")


def _hsum16(v):
    s = [v[i] for i in range(16)]
    while len(s) > 1:
        s = [s[2 * i] + s[2 * i + 1] for i in range(len(s) // 2)]
    return s[0]


def _hmax16(v):
    s = [v[i] for i in range(16)]
    while len(s) > 1:
        s = [jnp.maximum(s[2 * i], s[2 * i + 1]) for i in range(len(s) // 2)]
    return s[0]


def _p1_body(xl_hbm, xr_hbm, ep_hbm, src_hbm, dst_hbm, eid_hbm, att_hbm,
             alpha_hbm, srcv, dstv, eidv, attv, xlb0, xrb0, epb0, xlb1, xrb1,
             epb1, aloc, s10, s20, s30, s11, s21, s31):
    w = _wid()
    e0 = w * EPW
    pltpu.sync_copy(src_hbm.at[pl.ds(e0, EPW + 2 * G)], srcv)
    pltpu.sync_copy(dst_hbm.at[pl.ds(e0, EPW + 2 * G)], dstv)
    pltpu.sync_copy(eid_hbm.at[pl.ds(e0, EPW + 2 * G)], eidv)
    pltpu.sync_copy(att_hbm, attv)
    ri = lax.iota(jnp.int32, 16)
    bufs = ((xlb0, xrb0, epb0, s10, s20, s30),
            (xlb1, xrb1, epb1, s11, s21, s31))

    def issue(ch, bs):
        xlb, xrb, epb, s1, s2, s3 = bs
        off = ch * G
        pltpu.async_copy(xl_hbm.at[srcv.at[pl.ds(off, G)]], xlb, s1)
        pltpu.async_copy(xr_hbm.at[dstv.at[pl.ds(off, G)]], xrb, s2)
        pltpu.async_copy(ep_hbm.at[eidv.at[pl.ds(off, G)]], epb, s3)

    def process(ch, bs):
        xlb, xrb, epb, s1, s2, s3 = bs
        pltpu.make_async_copy(xl_hbm.at[srcv.at[pl.ds(0, G)]], xlb, s1).wait()
        pltpu.make_async_copy(xr_hbm.at[dstv.at[pl.ds(0, G)]], xrb, s2).wait()
        pltpu.make_async_copy(ep_hbm.at[eidv.at[pl.ds(0, G)]], epb, s3).wait()
        off = ch * G

        def edge(e, res):
            acc = [jnp.zeros((16,), jnp.float32) for _ in range(HEADS)]
            for c in range(NCH):
                sl = pl.ds(c * 16, 16)
                sv = xlb[e, sl] + xrb[e, sl] + epb[e, sl]
                m = jnp.maximum(sv, jnp.float32(0.2) * sv)
                h = c // 8
                av = attv[pl.ds(c * 16, 16)]
                acc[h] = acc[h] + m * av
            out = []
            for h in range(HEADS):
                out.append(jnp.where(ri == e, jnp.full((16,), _hsum16(acc[h])),
                                     res[h]))
            return tuple(out)

        z16 = jnp.zeros((16,), jnp.float32)
        res = lax.fori_loop(0, G, edge, (z16,) * HEADS)
        for h in range(HEADS):
            aloc[pl.ds(h * EPW + off, 16)] = res[h]

    issue(0, bufs[0])

    def step(t, _):
        issue(2 * t + 1, bufs[1])
        process(2 * t, bufs[0])
        issue(2 * t + 2, bufs[0])
        process(2 * t + 1, bufs[1])
        return 0

    lax.fori_loop(0, EPW // (2 * G), step, 0)
    process(EPW // G - 1, bufs[0])
    for h in range(HEADS):
        pltpu.sync_copy(aloc.at[pl.ds(h * EPW, EPW)],
                        alpha_hbm.at[pl.ds(h * E2P + e0, EPW)])


def _p1(xl, xr, ep, srcp, dstp, eidp, att):
    kfn = pl.kernel(
        _p1_body,
        out_type=jax.ShapeDtypeStruct((HEADS * E2P,), jnp.float32),
        mesh=_mesh,
        scratch_types=[
            pltpu.VMEM((EPW + 2 * G,), jnp.int32),
            pltpu.VMEM((EPW + 2 * G,), jnp.int32),
            pltpu.VMEM((EPW + 2 * G,), jnp.int32),
            pltpu.VMEM((D_HID,), jnp.float32),
            pltpu.VMEM((G, D_HID), jnp.float32),
            pltpu.VMEM((G, D_HID), jnp.float32),
            pltpu.VMEM((G, D_HID), jnp.float32),
            pltpu.VMEM((G, D_HID), jnp.float32),
            pltpu.VMEM((G, D_HID), jnp.float32),
            pltpu.VMEM((G, D_HID), jnp.float32),
            pltpu.VMEM((HEADS * EPW,), jnp.float32),
            pltpu.SemaphoreType.DMA,
            pltpu.SemaphoreType.DMA,
            pltpu.SemaphoreType.DMA,
            pltpu.SemaphoreType.DMA,
            pltpu.SemaphoreType.DMA,
            pltpu.SemaphoreType.DMA,
        ],
    )
    return kfn(xl, xr, ep, srcp, dstp, eidp, att)


def _p2_body(alpha_hbm, ro_hbm, amax_hbm, den_hbm, win, am2, dn2, ro_v):
    w = _wid()
    n0 = w * NPW
    pltpu.sync_copy(ro_hbm.at[pl.ds(n0, NPW + 16)], ro_v)
    ri = lax.iota(jnp.int32, 16)

    def head(h, _):
        def group(g, cur_wb):
            rv = ro_v[pl.ds(g * 16, 16)]
            rvn = ro_v[pl.ds(g * 16 + 16, 16)]
            wb_c = cur_wb
            for k in range(16):
                s = rv[k]
                e = rvn[0] if k == 15 else rv[k + 1]
                c0 = s // 16
                c1 = (e + 15) // 16

                def chk(c, carry):
                    m, d, wb = carry
                    nwb = (c * 16) // WCAP * WCAP

                    @pl.when(nwb != wb)
                    def _():
                        pltpu.sync_copy(
                            alpha_hbm.at[pl.ds(h * E2P + nwb, WCAP)], win)

                    loc = c * 16 - nwb
                    a = win[pl.ds(loc, 16)]
                    ids = c * 16 + ri
                    vm = (ids >= s) & (ids < e)
                    am = jnp.where(vm, a, NEG_INF)
                    mn = jnp.maximum(m, am)
                    scale = jnp.where(mn == NEG_INF, jnp.float32(1.0),
                                      jnp.exp(m - mn))
                    t = jnp.where(am == NEG_INF, jnp.float32(0.0),
                                  jnp.exp(am - mn))
                    return (mn, d * scale + t, nwb)

                m0 = jnp.full((16,), NEG_INF, jnp.float32)
                d0 = jnp.zeros((16,), jnp.float32)
                m, d, wb_c = lax.fori_loop(c0, c1, chk, (m0, d0, wb_c))
                mm = _hmax16(m)
                dv2 = jnp.where(m == NEG_INF, jnp.float32(0.0),
                                d * jnp.exp(m - jnp.full((16,), mm)))
                dd = _hsum16(dv2)
                n_loc = g * 16 + k
                am2[pl.ds(n_loc * 16, 16)] = jnp.full((16,), mm)
                dn2[pl.ds(n_loc * 16, 16)] = jnp.full((16,), dd)
            return wb_c

        lax.fori_loop(0, NPW // 16, group, jnp.int32(-(2 ** 30)))
        pltpu.sync_copy(am2,
                        amax_hbm.at[pl.ds((h * NPAD + n0) * 16, NPW * 16)])
        pltpu.sync_copy(dn2,
                        den_hbm.at[pl.ds((h * NPAD + n0) * 16, NPW * 16)])
        return 0

    lax.fori_loop(0, HEADS, head, 0)


def _p2(alpha, ro):
    kfn = pl.kernel(
        _p2_body,
        out_type=(jax.ShapeDtypeStruct((HEADS * NPAD * 16,), jnp.float32),
                  jax.ShapeDtypeStruct((HEADS * NPAD * 16,), jnp.float32)),
        mesh=_mesh,
        scratch_types=[
            pltpu.VMEM((WCAP,), jnp.float32),
            pltpu.VMEM((NPW * 16,), jnp.float32),
            pltpu.VMEM((NPW * 16,), jnp.float32),
            pltpu.VMEM((NPW + 16,), jnp.int32),
        ],
    )
    return kfn(alpha, ro)


NQ = 4
NQROW = NPW // NQ


def _p3_body(xl_hbm, alpha_hbm, amax_hbm, den_hbm, src_hbm, dst_hbm, ro_hbm,
             out_hbm, srcw, dstw, aw, rep_m, rep_d, acc, rows, wbuf, ro_v,
             sg):
    w = _wid()
    n0 = w * NPW
    pltpu.sync_copy(ro_hbm.at[pl.ds(n0, NPW + 16)], ro_v)
    for h in range(HEADS):
        pltpu.sync_copy(amax_hbm.at[pl.ds((h * NPAD + n0) * 16, NPW * 16)],
                        rep_m.at[pl.ds(h * NPW * 16, NPW * 16)])
        pltpu.sync_copy(den_hbm.at[pl.ds((h * NPAD + n0) * 16, NPW * 16)],
                        rep_d.at[pl.ds(h * NPW * 16, NPW * 16)])
    ri = lax.iota(jnp.int32, 16)

    def quarter(q, _):
        nb = n0 + q * NQROW

        def zrow(i, _):
            acc[pl.ds(i * 16, 16)] = jnp.zeros((16,), jnp.float32)
            return 0

        lax.fori_loop(0, NQROW * D_HID // 16, zrow, 0)
        s_h = ro_v[pl.ds(q * NQROW, 16)][0]
        e_h = ro_v[pl.ds(q * NQROW + NQROW, 16)][0]
        base = s_h // 16 * 16
        nch = (e_h - base + 15) // 16

        def chk(j, _):
            goff = base + j * 16

            @pl.when(j % (EW3 // 16) == 0)
            def _():
                pltpu.sync_copy(src_hbm.at[pl.ds(goff, EW3)], srcw)
                pltpu.sync_copy(dst_hbm.at[pl.ds(goff, EW3)], dstw)
                for hh in range(HEADS):
                    pltpu.sync_copy(alpha_hbm.at[pl.ds(hh * E2P + goff, EW3)],
                                    aw.at[pl.ds(hh * EW3, EW3)])

            loc = j % (EW3 // 16) * 16
            pltpu.async_copy(xl_hbm.at[srcw.at[pl.ds(loc, G)]], rows,
                             sg).wait()
            dstv = dstw[pl.ds(loc, 16)]
            avs = [aw[pl.ds(h * EW3 + loc, 16)] for h in range(HEADS)]
            z16 = jnp.zeros((16,), jnp.float32)
            tv = [z16] * HEADS
            dv = [jnp.full((16,), jnp.float32(1.0))] * HEADS
            for e in range(G):
                dl = jnp.clip(dstv[e] - n0, 0, NPW - 1)
                lane = ri == e
                for h in range(HEADS):
                    mrow = rep_m[pl.ds((h * NPW + dl) * 16, 16)]
                    drow = rep_d[pl.ds((h * NPW + dl) * 16, 16)]
                    tv[h] = jnp.where(lane,
                                      jnp.full((16,), avs[h][e] - mrow[0]),
                                      tv[h])
                    dv[h] = jnp.where(lane, jnp.full((16,), drow[0]), dv[h])
            for h in range(HEADS):
                wbuf[pl.ds(h * 16, 16)] = (
                    jnp.exp(tv[h]) / (dv[h] + jnp.float32(1e-16)))
            for e in range(G):
                gid = goff + e

                @pl.when((gid >= s_h) & (gid < e_h))
                def _(e=e):
                    da = dstv[e] - nb

                    def hloop(h, _):
                        wvec = wbuf[pl.ds(h * 16, 16)]
                        wf = jnp.full((16,), wvec[e])
                        for cc in range(C // 16):
                            off = da * D_HID + h * C + cc * 16
                            plsc.addupdate(
                                acc.at[pl.ds(off, 16)],
                                wf * rows[e, pl.ds(h * C + cc * 16, 16)])
                        return 0

                    lax.fori_loop(0, HEADS, hloop, 0)
            return 0

        lax.fori_loop(0, nch, chk, 0)
        pltpu.sync_copy(
            acc, out_hbm.at[pl.ds(nb * D_HID, NQROW * D_HID)])
        return 0

    lax.fori_loop(0, NQ, quarter, 0)


def _p3(xl, alpha, amax, den, srcp, dstp, ro):
    kfn = pl.kernel(
        _p3_body,
        out_type=jax.ShapeDtypeStruct((NPAD * D_HID,), jnp.float32),
        mesh=_mesh,
        scratch_types=[
            pltpu.VMEM((EW3,), jnp.int32),
            pltpu.VMEM((EW3,), jnp.int32),
            pltpu.VMEM((HEADS * EW3,), jnp.float32),
            pltpu.VMEM((HEADS * NPW * 16,), jnp.float32),
            pltpu.VMEM((HEADS * NPW * 16,), jnp.float32),
            pltpu.VMEM((NQROW * D_HID,), jnp.float32),
            pltpu.VMEM((G, D_HID), jnp.float32),
            pltpu.VMEM((HEADS * 16,), jnp.float32),
            pltpu.VMEM((NPW + 16,), jnp.int32),
            pltpu.SemaphoreType.DMA,
        ],
    )
    return kfn(xl, alpha, amax, den, srcp, dstp, ro)


def _ln_block(x, g, b):
    mu = jnp.mean(x, axis=-1, keepdims=True)
    var = jnp.mean((x - mu) ** 2, axis=-1, keepdims=True)
    return (x - mu) * lax.rsqrt(var + 1e-5) * g + b


def _tc_pre_body(bf, g, b, Win, bin_, Wl, bl, Wr, br, x0o, xlo, xro):
    xn = _ln_block(bf[...], g[...], b[...])
    x0 = jnp.maximum(
        jnp.dot(xn, Win[...], preferred_element_type=jnp.float32) + bin_[...],
        0.0)
    x0o[...] = x0
    xlo[...] = jnp.dot(x0, Wl[...], preferred_element_type=jnp.float32) + bl[...]
    xro[...] = jnp.dot(x0, Wr[...], preferred_element_type=jnp.float32) + br[...]


def _tc_pre(bf, g, b, Win, bin_, Wl, bl, Wr, br):
    blk = 1000
    vec = lambda dim: pl.BlockSpec((dim,), lambda i: (0,))
    mat = lambda r, c: pl.BlockSpec((r, c), lambda i: (0, 0))
    out = jax.ShapeDtypeStruct((N, D_HID), jnp.float32)
    return pl.pallas_call(
        _tc_pre_body,
        grid=(N // blk,),
        in_specs=[
            pl.BlockSpec((blk, D_BLOCK), lambda i: (i, 0)),
            vec(D_BLOCK), vec(D_BLOCK),
            mat(D_BLOCK, D_HID), vec(D_HID),
            mat(D_HID, D_HID), vec(D_HID),
            mat(D_HID, D_HID), vec(D_HID),
        ],
        out_specs=[pl.BlockSpec((blk, D_HID), lambda i: (i, 0))] * 3,
        out_shape=[out, out, out],
    )(bf, g, b, Win, bin_, Wl, bl, Wr, br)


def _tc_ew_body(ea, We, be, eo):
    eo[...] = jnp.maximum(
        jnp.dot(ea[...], We[...], preferred_element_type=jnp.float32) + be[...],
        0.0)


def _tc_ew(ea, We, be):
    blk = 4000
    return pl.pallas_call(
        _tc_ew_body,
        grid=(E // blk,),
        in_specs=[
            pl.BlockSpec((blk, 16), lambda i: (i, 0)),
            pl.BlockSpec((16, D_HID), lambda i: (0, 0)),
            pl.BlockSpec((D_HID,), lambda i: (0,)),
        ],
        out_specs=pl.BlockSpec((blk, D_HID), lambda i: (i, 0)),
        out_shape=jax.ShapeDtypeStruct((E, D_HID), jnp.float32),
    )(ea, We, be)


def _tc_ep_body(ew, W1, W2, o1, o2):
    e = ew[...]
    o1[...] = jnp.dot(e, W1[...], preferred_element_type=jnp.float32)
    o2[...] = jnp.dot(e, W2[...], preferred_element_type=jnp.float32)


def _tc_ep(ew, W1, W2):
    blk = 2000
    out = jax.ShapeDtypeStruct((E, D_HID), jnp.float32)
    return pl.pallas_call(
        _tc_ep_body,
        grid=(E // blk,),
        in_specs=[
            pl.BlockSpec((blk, D_HID), lambda i: (i, 0)),
            pl.BlockSpec((D_HID, D_HID), lambda i: (0, 0)),
            pl.BlockSpec((D_HID, D_HID), lambda i: (0, 0)),
        ],
        out_specs=[pl.BlockSpec((blk, D_HID), lambda i: (i, 0))] * 2,
        out_shape=[out, out],
    )(ew, W1, W2)


def _tc_mid_body(x, msg, bo, g, b, Wl, bl, Wr, br, x1o, xlo, xro):
    xn = _ln_block(x[...] + msg[...] + bo[...], g[...], b[...])
    x1o[...] = xn
    xlo[...] = jnp.dot(xn, Wl[...], preferred_element_type=jnp.float32) + bl[...]
    xro[...] = jnp.dot(xn, Wr[...], preferred_element_type=jnp.float32) + br[...]


def _tc_mid(x, msg, bo, g, b, Wl, bl, Wr, br):
    blk = 1000
    vec = lambda dim: pl.BlockSpec((dim,), lambda i: (0,))
    mat = lambda r, c: pl.BlockSpec((r, c), lambda i: (0, 0))
    out = jax.ShapeDtypeStruct((N, D_HID), jnp.float32)
    return pl.pallas_call(
        _tc_mid_body,
        grid=(N // blk,),
        in_specs=[
            pl.BlockSpec((blk, D_HID), lambda i: (i, 0)),
            pl.BlockSpec((blk, D_HID), lambda i: (i, 0)),
            vec(D_HID), vec(D_HID), vec(D_HID),
            mat(D_HID, D_HID), vec(D_HID),
            mat(D_HID, D_HID), vec(D_HID),
        ],
        out_specs=[pl.BlockSpec((blk, D_HID), lambda i: (i, 0))] * 3,
        out_shape=[out, out, out],
    )(x, msg, bo, g, b, Wl, bl, Wr, br)


def _tc_out_body(x, msg, bo, g, b, W1, b1, W2, b2, zo):
    xn = _ln_block(x[...] + msg[...] + bo[...], g[...], b[...])
    hh = jnp.maximum(
        jnp.dot(xn, W1[...], preferred_element_type=jnp.float32) + b1[...],
        0.0)
    zo[...] = jnp.dot(hh, W2[...], preferred_element_type=jnp.float32) + b2[...]


def _tc_out(x, msg, bo, g, b, W1, b1, W2, b2):
    blk = 1000
    vec = lambda dim: pl.BlockSpec((dim,), lambda i: (0,))
    mat = lambda r, c: pl.BlockSpec((r, c), lambda i: (0, 0))
    return pl.pallas_call(
        _tc_out_body,
        grid=(N // blk,),
        in_specs=[
            pl.BlockSpec((blk, D_HID), lambda i: (i, 0)),
            pl.BlockSpec((blk, D_HID), lambda i: (i, 0)),
            vec(D_HID), vec(D_HID), vec(D_HID),
            mat(D_HID, D_Z), vec(D_Z),
            mat(D_Z, D_Z), vec(D_Z),
        ],
        out_specs=pl.BlockSpec((blk, D_Z), lambda i: (i, 0)),
        out_shape=jax.ShapeDtypeStruct((N, D_Z), jnp.float32),
    )(x, msg, bo, g, b, W1, b1, W2, b2)


def kernel(block_features, block_edge_index, block_edge_attr, ln_in_g,
           ln_in_b, W_in, b_in, W_e, b_e, Wl1, bl1, Wr1, br1, We1, att1, bo1,
           ln1_g, ln1_b, Wl2, bl2, Wr2, br2, We2, att2, bo2, ln2_g, ln2_b,
           Wo1, bo1w, Wo2, bo2w):
    ei = block_edge_index.astype(jnp.int32)
    s, d = ei[0], ei[1]
    src2 = jnp.concatenate([s, d])
    dst2 = jnp.concatenate([d, s])
    eid = jnp.arange(E, dtype=jnp.int32)
    eid2 = jnp.concatenate([eid, eid])
    perm = jnp.argsort(dst2)
    ssrc = src2[perm]
    sdst = dst2[perm]
    seid = eid2[perm]
    pad = E2P - E2
    ssrc_p = jnp.pad(ssrc, (0, pad))
    sdst_p = jnp.pad(sdst, (0, pad))
    seid_p = jnp.pad(seid, (0, pad))
    ro = jnp.searchsorted(sdst, jnp.arange(NPAD + 1, dtype=jnp.int32),
                          side="left").astype(jnp.int32)
    ro_p = jnp.pad(ro, (0, 15))

    x0, xl1, xr1 = _tc_pre(block_features, ln_in_g, ln_in_b, W_in, b_in,
                           Wl1, bl1, Wr1, br1)
    ew = _tc_ew(block_edge_attr, W_e, b_e)
    ep1, ep2 = _tc_ep(ew, We1, We2)

    att1f = jnp.reshape(att1, (D_HID,))
    att2f = jnp.reshape(att2, (D_HID,))
    al1 = _p1(xl1, xr1, ep1, ssrc_p, sdst_p, seid_p, att1f)
    am1, dn1 = _p2(al1, ro_p)
    msg1 = jnp.reshape(_p3(xl1, al1, am1, dn1, ssrc_p, sdst_p,
                           ro_p), (NPAD, D_HID))

    x1, xl2, xr2 = _tc_mid(x0, msg1, bo1, ln1_g, ln1_b, Wl2, bl2, Wr2, br2)

    al2 = _p1(xl2, xr2, ep2, ssrc_p, sdst_p, seid_p, att2f)
    am2, dn2 = _p2(al2, ro_p)
    msg2 = jnp.reshape(_p3(xl2, al2, am2, dn2, ssrc_p, sdst_p,
                           ro_p), (NPAD, D_HID))

    return _tc_out(x1, msg2, bo2, ln2_g, ln2_b, Wo1, bo1w, Wo2, bo2w)
